# Initial kernel scaffold; baseline (speedup 1.0000x reference)
import jax
import jax.numpy as jnp
from jax.experimental import pallas as pl


def kernel(node_mrna, node_cnv, pooled_mrna, pooled_cnv, gene_batch):
    raise NotImplementedError("write your pallas kernel here")


# speedup vs baseline: 12.3001x; 12.3001x over previous
import jax
import jax.numpy as jnp
from jax.experimental import pallas as pl
from jax.experimental.pallas import tpu as pltpu

N, B, H = 100000, 256, 512
R = 2000
NUM_BLOCKS = N // R


def _loss_body(ids_ref, nm_ref, nc_ref, pm_ref, pc_ref,
               o_tot_ref, o_m_ref, o_c_ref,
               acc_m, acc_c, cnt):
    i = pl.program_id(0)

    @pl.when(i == 0)
    def _init():
        acc_m[...] = jnp.zeros_like(acc_m)
        acc_c[...] = jnp.zeros_like(acc_c)
        cnt[...] = jnp.zeros_like(cnt)

    ids = ids_ref[0, 0, :]
    seg = jax.lax.broadcasted_iota(jnp.int32, (B, R), 0)
    onehot = (seg == ids[None, :]).astype(jnp.float32)

    acc_m[...] += jnp.dot(onehot, nm_ref[...],
                          preferred_element_type=jnp.float32)
    acc_c[...] += jnp.dot(onehot, nc_ref[...],
                          preferred_element_type=jnp.float32)
    cnt[...] += jnp.sum(onehot, axis=1, keepdims=True)

    @pl.when(i == NUM_BLOCKS - 1)
    def _finish():
        inv = 1.0 / jnp.maximum(cnt[...], 1.0)
        mean_m = acc_m[...] * inv
        mean_c = acc_c[...] * inv

        def cos_dist_mean(a, b):
            eps = 1e-8
            num = jnp.sum(a * b, axis=1, keepdims=True)
            na = jnp.maximum(jnp.sqrt(jnp.sum(a * a, axis=1, keepdims=True)), eps)
            nb = jnp.maximum(jnp.sqrt(jnp.sum(b * b, axis=1, keepdims=True)), eps)
            cos = num / (na * nb)
            return jnp.mean(1.0 - cos)

        lm = cos_dist_mean(mean_m, pm_ref[...])
        lc = cos_dist_mean(mean_c, pc_ref[...])
        o_m_ref[...] = jnp.reshape(lm, (1, 1))
        o_c_ref[...] = jnp.reshape(lc, (1, 1))
        o_tot_ref[...] = jnp.reshape((lm + lc) * 0.5, (1, 1))


def kernel(node_mrna, node_cnv, pooled_mrna, pooled_cnv, gene_batch):
    ids3 = gene_batch.reshape(NUM_BLOCKS, 1, R)
    scalar = jax.ShapeDtypeStruct((1, 1), jnp.float32)
    tot, lm, lc = pl.pallas_call(
        _loss_body,
        grid=(NUM_BLOCKS,),
        in_specs=[
            pl.BlockSpec((1, 1, R), lambda i: (i, 0, 0)),
            pl.BlockSpec((R, H), lambda i: (i, 0)),
            pl.BlockSpec((R, H), lambda i: (i, 0)),
            pl.BlockSpec((B, H), lambda i: (0, 0)),
            pl.BlockSpec((B, H), lambda i: (0, 0)),
        ],
        out_specs=[
            pl.BlockSpec((1, 1), lambda i: (0, 0)),
            pl.BlockSpec((1, 1), lambda i: (0, 0)),
            pl.BlockSpec((1, 1), lambda i: (0, 0)),
        ],
        out_shape=[scalar, scalar, scalar],
        scratch_shapes=[
            pltpu.VMEM((B, H), jnp.float32),
            pltpu.VMEM((B, H), jnp.float32),
            pltpu.VMEM((B, 1), jnp.float32),
        ],
    )(ids3, node_mrna, node_cnv, pooled_mrna, pooled_cnv)
    return (tot[0, 0], lm[0, 0], lc[0, 0])


# speedup vs baseline: 12.8333x; 1.0434x over previous
import jax
import jax.numpy as jnp
from jax.experimental import pallas as pl
from jax.experimental.pallas import tpu as pltpu

N, B, H = 100000, 256, 512
R = 4000
NUM_BLOCKS = N // R


def _loss_body(ids_ref, nm_ref, nc_ref, pm_ref, pc_ref,
               o_tot_ref, o_m_ref, o_c_ref,
               acc_m, acc_c, cnt):
    i = pl.program_id(0)

    @pl.when(i == 0)
    def _init():
        acc_m[...] = jnp.zeros_like(acc_m)
        acc_c[...] = jnp.zeros_like(acc_c)
        cnt[...] = jnp.zeros_like(cnt)

    ids = ids_ref[0, 0, :]
    seg = jax.lax.broadcasted_iota(jnp.int32, (B, R), 0)
    onehot = (seg == ids[None, :]).astype(jnp.float32)

    acc_m[...] += jnp.dot(onehot, nm_ref[...],
                          preferred_element_type=jnp.float32)
    acc_c[...] += jnp.dot(onehot, nc_ref[...],
                          preferred_element_type=jnp.float32)
    cnt[...] += jnp.sum(onehot, axis=1, keepdims=True)

    @pl.when(i == NUM_BLOCKS - 1)
    def _finish():
        inv = 1.0 / jnp.maximum(cnt[...], 1.0)
        mean_m = acc_m[...] * inv
        mean_c = acc_c[...] * inv

        def cos_dist_mean(a, b):
            eps = 1e-8
            num = jnp.sum(a * b, axis=1, keepdims=True)
            na = jnp.maximum(jnp.sqrt(jnp.sum(a * a, axis=1, keepdims=True)), eps)
            nb = jnp.maximum(jnp.sqrt(jnp.sum(b * b, axis=1, keepdims=True)), eps)
            cos = num / (na * nb)
            return jnp.mean(1.0 - cos)

        lm = cos_dist_mean(mean_m, pm_ref[...])
        lc = cos_dist_mean(mean_c, pc_ref[...])
        o_m_ref[...] = jnp.reshape(lm, (1, 1))
        o_c_ref[...] = jnp.reshape(lc, (1, 1))
        o_tot_ref[...] = jnp.reshape((lm + lc) * 0.5, (1, 1))


def kernel(node_mrna, node_cnv, pooled_mrna, pooled_cnv, gene_batch):
    ids3 = gene_batch.reshape(NUM_BLOCKS, 1, R)
    scalar = jax.ShapeDtypeStruct((1, 1), jnp.float32)
    tot, lm, lc = pl.pallas_call(
        _loss_body,
        grid=(NUM_BLOCKS,),
        in_specs=[
            pl.BlockSpec((1, 1, R), lambda i: (i, 0, 0)),
            pl.BlockSpec((R, H), lambda i: (i, 0)),
            pl.BlockSpec((R, H), lambda i: (i, 0)),
            pl.BlockSpec((B, H), lambda i: (0, 0)),
            pl.BlockSpec((B, H), lambda i: (0, 0)),
        ],
        out_specs=[
            pl.BlockSpec((1, 1), lambda i: (0, 0)),
            pl.BlockSpec((1, 1), lambda i: (0, 0)),
            pl.BlockSpec((1, 1), lambda i: (0, 0)),
        ],
        out_shape=[scalar, scalar, scalar],
        scratch_shapes=[
            pltpu.VMEM((B, H), jnp.float32),
            pltpu.VMEM((B, H), jnp.float32),
            pltpu.VMEM((B, 1), jnp.float32),
        ],
    )(ids3, node_mrna, node_cnv, pooled_mrna, pooled_cnv)
    return (tot[0, 0], lm[0, 0], lc[0, 0])
